# baseline (speedup 1.0000x reference)
import functools

import jax
import jax.numpy as jnp
from jax import lax
from jax.experimental import pallas as pl
from jax.experimental.pallas import tpu as pltpu
from jax.experimental.pallas import tpu_sc as plsc

N = 10000
E = 160000
H = 256
G = 64

NSUB = 16
CHUNK = 64
NCHUNK = 160
SUPER = 40
EPS = CHUNK * NCHUNK
PADC = 144
ROWS_PER_SUB = 624
NEG = -1e30


def _mm_kernel(x_ref, w_ref, o_ref):
    o_ref[...] = jnp.dot(x_ref[...], w_ref[...],
                         preferred_element_type=jnp.float32)


def _mm(x, w, bm=2000):
    M, K = x.shape
    _, Nc = w.shape
    return pl.pallas_call(
        _mm_kernel,
        grid=(M // bm,),
        in_specs=[
            pl.BlockSpec((bm, K), lambda i: (i, 0)),
            pl.BlockSpec((K, Nc), lambda i: (0, 0)),
        ],
        out_specs=pl.BlockSpec((bm, Nc), lambda i: (i, 0)),
        out_shape=jax.ShapeDtypeStruct((M, Nc), jnp.float32),
    )(x, w)


def _pad16(bm, acc_sub, svec):
    it = lax.broadcasted_iota(jnp.int32, (bm, 16), 1)
    pad = jnp.where(it == 0, 1.0, 0.0) + jnp.where(it == 1, 1.0, 0.0) * svec
    return jnp.concatenate([acc_sub, pad], axis=1)


def _tcb_kernel(x_ref, w_ref, b_ref, lo_ref, hi_ref, sd_ref, *, bm):
    acc = jnp.dot(x_ref[...], w_ref[...],
                  preferred_element_type=jnp.float32) + b_ref[0:1, :]
    sv = acc[:, 256:257]
    lo_ref[...] = _pad16(bm, acc[:, 0:128], sv)
    hi_ref[...] = _pad16(bm, acc[:, 128:256], sv)
    sd_ref[...] = acc[:, 256:384]


def _tc_in(x, Wcomb, biasrow, bm=2000):
    M = x.shape[0]
    return pl.pallas_call(
        functools.partial(_tcb_kernel, bm=bm),
        grid=(M // bm,),
        in_specs=[
            pl.BlockSpec((bm, H), lambda i: (i, 0)),
            pl.BlockSpec((H, H + 128), lambda i: (0, 0)),
            pl.BlockSpec((8, H + 128), lambda i: (0, 0)),
        ],
        out_specs=[
            pl.BlockSpec((bm, PADC), lambda i: (i, 0)),
            pl.BlockSpec((bm, PADC), lambda i: (i, 0)),
            pl.BlockSpec((bm, 128), lambda i: (i, 0)),
        ],
        out_shape=[
            jax.ShapeDtypeStruct((M, PADC), jnp.float32),
            jax.ShapeDtypeStruct((M, PADC), jnp.float32),
            jax.ShapeDtypeStruct((M, 128), jnp.float32),
        ],
    )(x, Wcomb, biasrow)


def _normalize(a_ref):
    lo = a_ref[0, :, 0:128] / (a_ref[0, :, 128:129] + 1e-16)
    hi = a_ref[1, :, 0:128] / (a_ref[1, :, 128:129] + 1e-16)
    return jnp.concatenate([lo, hi], axis=1)


def _tcc_kernel(a_ref, bp_ref, w_ref, b_ref, lo_ref, hi_ref, sd_ref, *, bm):
    h = jax.nn.relu(_normalize(a_ref) + bp_ref[0:1, :])
    acc = jnp.dot(h, w_ref[...],
                  preferred_element_type=jnp.float32) + b_ref[0:1, :]
    sv = acc[:, 256:257]
    lo_ref[...] = _pad16(bm, acc[:, 0:128], sv)
    hi_ref[...] = _pad16(bm, acc[:, 128:256], sv)
    sd_ref[...] = acc[:, 256:384]


def _tcc2_kernel(a_ref, bp_ref, w_ref, lo1_ref, hi1_ref, sd1_ref,
                 lo2_ref, hi2_ref, sd2_ref, *, bm):
    h = jax.nn.relu(_normalize(a_ref) + bp_ref[0:1, :])
    acc = jnp.dot(h, w_ref[...], preferred_element_type=jnp.float32)
    sv1 = acc[:, 256:257]
    lo1_ref[...] = _pad16(bm, acc[:, 0:128], sv1)
    hi1_ref[...] = _pad16(bm, acc[:, 128:256], sv1)
    sd1_ref[...] = acc[:, 256:384]
    sv2 = acc[:, 640:641]
    lo2_ref[...] = _pad16(bm, acc[:, 384:512], sv2)
    hi2_ref[...] = _pad16(bm, acc[:, 512:640], sv2)
    sd2_ref[...] = acc[:, 640:768]


def _tc_mid2(acc, bprevrow, Wcomb2, bm=2000):
    outs = [
        jax.ShapeDtypeStruct((N, PADC), jnp.float32),
        jax.ShapeDtypeStruct((N, PADC), jnp.float32),
        jax.ShapeDtypeStruct((N, 128), jnp.float32),
    ] * 2
    return pl.pallas_call(
        functools.partial(_tcc2_kernel, bm=bm),
        grid=(N // bm,),
        in_specs=[
            pl.BlockSpec((2, bm, PADC), lambda i: (0, i, 0)),
            pl.BlockSpec((8, H), lambda i: (0, 0)),
            pl.BlockSpec((H, 768), lambda i: (0, 0)),
        ],
        out_specs=[pl.BlockSpec((bm, PADC), lambda i: (i, 0)),
                   pl.BlockSpec((bm, PADC), lambda i: (i, 0)),
                   pl.BlockSpec((bm, 128), lambda i: (i, 0))] * 2,
        out_shape=outs,
    )(acc, bprevrow, Wcomb2)


def _tc_mid(acc, bprevrow, Wcomb, biasrow, bm=2000):
    return pl.pallas_call(
        functools.partial(_tcc_kernel, bm=bm),
        grid=(N // bm,),
        in_specs=[
            pl.BlockSpec((2, bm, PADC), lambda i: (0, i, 0)),
            pl.BlockSpec((8, H), lambda i: (0, 0)),
            pl.BlockSpec((H, H + 128), lambda i: (0, 0)),
            pl.BlockSpec((8, H + 128), lambda i: (0, 0)),
        ],
        out_specs=[
            pl.BlockSpec((bm, PADC), lambda i: (i, 0)),
            pl.BlockSpec((bm, PADC), lambda i: (i, 0)),
            pl.BlockSpec((bm, 128), lambda i: (i, 0)),
        ],
        out_shape=[
            jax.ShapeDtypeStruct((N, PADC), jnp.float32),
            jax.ShapeDtypeStruct((N, PADC), jnp.float32),
            jax.ShapeDtypeStruct((N, 128), jnp.float32),
        ],
    )(acc, bprevrow, Wcomb, biasrow)


def _tcd_kernel(a1_ref, a2_ref, bp1_ref, bp2_ref, wp1_ref, wp2_ref,
                bprj_ref, batch_ref, o_ref, sums, *, bm, ng):
    i = pl.program_id(0)
    xg1 = _normalize(a1_ref) + bp1_ref[0:1, :]
    xg2 = _normalize(a2_ref) + bp2_ref[0:1, :]
    z1 = jnp.dot(xg1, wp1_ref[...],
                 preferred_element_type=jnp.float32)[:, 0:1] + bprj_ref[0:1, 0:1]
    z2 = jnp.dot(xg2, wp2_ref[...],
                 preferred_element_type=jnp.float32)[:, 0:1] + bprj_ref[0:1, 1:2]
    xw1 = jax.nn.sigmoid(z1) * xg1
    xw2 = jax.nn.sigmoid(z2) * xg2
    b = batch_ref[0, 0, :]
    oh = (b[:, None] == lax.broadcasted_iota(jnp.int32, (bm, G), 1)
          ).astype(jnp.float32)
    payload = jnp.concatenate(
        [xw1, xw2, jnp.ones((bm, 128), jnp.float32)], axis=1)
    contrib = lax.dot_general(oh, payload, (((0,), (0,)), ((), ())),
                              preferred_element_type=jnp.float32)

    @pl.when(i == 0)
    def _():
        sums[...] = jnp.zeros_like(sums)

    sums[...] += contrib

    @pl.when(i == ng - 1)
    def _():
        cnt = jnp.clip(sums[:, 512:513], 1.0)
        o_ref[...] = sums[:, 0:512] / cnt


def _tc_pool(acc1, acc2, bp1row, bp2row, Wp1pad, Wp2pad, bprjrow, batch3d,
             bm=1000):
    ng = N // bm
    return pl.pallas_call(
        functools.partial(_tcd_kernel, bm=bm, ng=ng),
        grid=(ng,),
        in_specs=[
            pl.BlockSpec((2, bm, PADC), lambda i: (0, i, 0)),
            pl.BlockSpec((2, bm, PADC), lambda i: (0, i, 0)),
            pl.BlockSpec((8, H), lambda i: (0, 0)),
            pl.BlockSpec((8, H), lambda i: (0, 0)),
            pl.BlockSpec((H, 128), lambda i: (0, 0)),
            pl.BlockSpec((H, 128), lambda i: (0, 0)),
            pl.BlockSpec((8, 128), lambda i: (0, 0)),
            pl.BlockSpec((1, 1, bm), lambda i: (i, 0, 0)),
        ],
        out_specs=pl.BlockSpec((G, 512), lambda i: (0, 0)),
        out_shape=jax.ShapeDtypeStruct((G, 512), jnp.float32),
        scratch_shapes=[pltpu.VMEM((G, 640), jnp.float32)],
    )(acc1, acc2, bp1row, bp2row, Wp1pad, Wp2pad, bprjrow, batch3d)


def _sc_body(lo, hi, d_hbm, src3d, dst3d, te3d, out,
             acc, dtab, srcc, dstc, tec, rows0, rows1, pb,
             g0, g1, s0, s1):
    c = lax.axis_index("c")
    s = lax.axis_index("s")

    def zrow(k, _):
        for j in range(PADC // 16):
            rows0[k, pl.ds(j * 16, 16)] = jnp.zeros((16,), jnp.float32)
        return 0
    lax.fori_loop(0, CHUNK, zrow, 0)
    base = s * ROWS_PER_SUB
    for k in range(9):
        pltpu.sync_copy(rows0, acc.at[pl.ds(base + k * CHUNK, CHUNK)])
    pltpu.sync_copy(rows0.at[pl.ds(0, 48)], acc.at[pl.ds(base + 576, 48)])

    @pl.when(s == NSUB - 1)
    def _():
        pltpu.sync_copy(rows0.at[pl.ds(0, 16)],
                        acc.at[pl.ds(NSUB * ROWS_PER_SUB, 16)])

    plsc.subcore_barrier()

    pltpu.sync_copy(d_hbm, dtab)

    def _gather(idx_ref, buf, sem):
        @pl.when(c == 0)
        def _():
            pltpu.async_copy(lo.at[idx_ref], buf, sem)

        @pl.when(c == 1)
        def _():
            pltpu.async_copy(hi.at[idx_ref], buf, sem)

    def _drain(buf, sem):
        pltpu.make_async_copy(lo.at[srcc.at[0]], buf, sem).wait()

    def _process(row, buf, gsem, ssem):
        _drain(buf, gsem)

        col129 = jnp.full((16,), 129, jnp.int32)
        for g in range(CHUNK // 16):
            sl = pl.ds(g * 16, 16)
            rowi = lax.iota(jnp.int32, 16) + g * 16
            sv = plsc.load_gather(buf, [rowi, col129])
            dv = plsc.load_gather(dtab, [dstc[row, sl]])
            alpha = sv + dv + tec[row, sl]
            alpha = jnp.where(alpha > 0, alpha, 0.2 * alpha)
            pb[sl] = jnp.exp(alpha)

        @plsc.parallel_loop(0, CHUNK, unroll=8)
        def _(e):
            pv = plsc.load_gather(pb, [jnp.full((16,), e, jnp.int32)])
            for j in range(PADC // 16):
                jsl = pl.ds(j * 16, 16)
                buf[e, jsl] = buf[e, jsl] * pv

        pltpu.async_copy(buf, acc.at[dstc.at[row]], ssem, add=True)

    def pair_body(p, _):
        sb = p // (SUPER // 2)

        @pl.when(p > 0)
        def _():
            pltpu.make_async_copy(rows0, acc.at[dstc.at[0]], s0).wait()
            pltpu.make_async_copy(rows1, acc.at[dstc.at[0]], s1).wait()

        @pl.when(p % (SUPER // 2) == 0)
        def _():
            cb = sb * SUPER
            pltpu.sync_copy(src3d.at[s].at[pl.ds(cb, SUPER)], srcc)
            pltpu.sync_copy(dst3d.at[s].at[pl.ds(cb, SUPER)], dstc)
            pltpu.sync_copy(te3d.at[s].at[pl.ds(cb, SUPER)], tec)

        kk = (p % (SUPER // 2)) * 2

        _gather(srcc.at[kk], rows0, g0)
        _gather(srcc.at[kk + 1], rows1, g1)
        _process(kk, rows0, g0, s0)
        _process(kk + 1, rows1, g1, s1)
        return 0

    lax.fori_loop(0, NCHUNK // 2, pair_body, 0)
    pltpu.make_async_copy(rows0, acc.at[dstc.at[0]], s0).wait()
    pltpu.make_async_copy(rows1, acc.at[dstc.at[0]], s1).wait()
    plsc.subcore_barrier()

    pltpu.sync_copy(acc.at[pl.ds(base, ROWS_PER_SUB)],
                    out.at[c].at[pl.ds(base, ROWS_PER_SUB)])

    @pl.when(s == NSUB - 1)
    def _():
        pltpu.sync_copy(acc.at[pl.ds(NSUB * ROWS_PER_SUB, 16)],
                        out.at[c].at[pl.ds(NSUB * ROWS_PER_SUB, 16)])


_sc_mesh = plsc.VectorSubcoreMesh(core_axis_name="c", subcore_axis_name="s")

_sc_gat = pl.kernel(
    _sc_body,
    out_type=jax.ShapeDtypeStruct((2, N, PADC), jnp.float32),
    mesh=_sc_mesh,
    compiler_params=pltpu.CompilerParams(use_tc_tiling_on_sc=False,
                                         needs_layout_passes=False),
    scratch_types=[
        pltpu.VMEM_SHARED((N, PADC), jnp.float32),
        pltpu.VMEM((N,), jnp.float32),
        pltpu.VMEM((SUPER, CHUNK), jnp.int32),
        pltpu.VMEM((SUPER, CHUNK), jnp.int32),
        pltpu.VMEM((SUPER, CHUNK), jnp.float32),
        pltpu.VMEM((CHUNK, PADC), jnp.float32),
        pltpu.VMEM((CHUNK, PADC), jnp.float32),
        pltpu.VMEM((CHUNK,), jnp.float32),
        pltpu.SemaphoreType.DMA,
        pltpu.SemaphoreType.DMA,
        pltpu.SemaphoreType.DMA,
        pltpu.SemaphoreType.DMA,
    ],
)


def _aug(W, a_s, a_d):
    Z = jnp.zeros((H, 128), jnp.float32)
    Z = Z.at[:, 0].set(W @ a_s)
    Z = Z.at[:, 1].set(W @ a_d)
    return jnp.concatenate([W, Z], axis=1)


def _row8(v):
    return jnp.broadcast_to(v[None, :], (8, v.shape[0]))


def _slab_f32(v):
    v2 = v.reshape(NSUB, N)
    v2 = jnp.concatenate(
        [v2, jnp.full((NSUB, EPS - N), NEG, jnp.float32)], axis=1)
    return v2.reshape(NSUB, NCHUNK, CHUNK)


def _slab_i32(v):
    v2 = v.reshape(NSUB, N)
    v2 = jnp.concatenate(
        [v2, jnp.zeros((NSUB, EPS - N), jnp.int32)], axis=1)
    return v2.reshape(NSUB, NCHUNK, CHUNK)


def kernel(x, edge_index, edge_attr, batch, W_np, b_np, W_ep, b_ep, W1, a_s1, a_d1, We1, a_e1, b1, W2, a_s2, a_d2, We2, a_e2, b2, Wp1, as_p1, ad_p1, bp1, Wproj1, bproj1, Wp2, as_p2, ad_p2, bp2, Wproj2, bproj2):
    src, dst = edge_index[0], edge_index[1]
    src2d = _slab_i32(src)
    dst2d = _slab_i32(dst)

    v1 = We1 @ a_e1
    v2 = We2 @ a_e2
    Vfold = jnp.zeros((16, 128), jnp.float32)
    Vfold = Vfold.at[:, 0].set(W_ep @ v1)
    Vfold = Vfold.at[:, 1].set(W_ep @ v2)
    te_both = _mm(edge_attr, Vfold)
    te1_2d = _slab_f32(te_both[:, 0])
    te2_2d = _slab_f32(te_both[:, 1])
    te0_2d = _slab_f32(jnp.zeros((E,), jnp.float32))

    W1aug = _aug(W1, a_s1, a_d1)
    Wcomb1 = W_np @ W1aug
    bias1 = (b_np @ W1aug).at[H + 1].add(b_ep @ v1)
    lo1, hi1, sd1 = _tc_in(x, Wcomb1, _row8(bias1))
    acc1 = _sc_gat(lo1, hi1, sd1[:, 1], src2d, dst2d, te1_2d)

    bias2 = jnp.zeros((H + 128,), jnp.float32).at[H + 1].add(b_ep @ v2)
    lo2, hi2, sd2 = _tc_mid(acc1, _row8(b1), _aug(W2, a_s2, a_d2),
                            _row8(bias2))
    acc2 = _sc_gat(lo2, hi2, sd2[:, 1], src2d, dst2d, te2_2d)

    Wcomb2 = jnp.concatenate(
        [_aug(Wp1, as_p1, ad_p1), _aug(Wp2, as_p2, ad_p2)], axis=1)
    loP1, hiP1, sdP1, loP2, hiP2, sdP2 = _tc_mid2(acc2, _row8(b2), Wcomb2)
    accP1 = _sc_gat(loP1, hiP1, sdP1[:, 1], src2d, dst2d, te0_2d)
    accP2 = _sc_gat(loP2, hiP2, sdP2[:, 1], src2d, dst2d, te0_2d)

    Wp1pad = jnp.zeros((H, 128), jnp.float32).at[:, 0].set(Wproj1[:, 0])
    Wp2pad = jnp.zeros((H, 128), jnp.float32).at[:, 0].set(Wproj2[:, 0])
    bprj = jnp.zeros((8, 128), jnp.float32)
    bprj = bprj.at[:, 0].set(bproj1[0]).at[:, 1].set(bproj2[0])
    batch3d = batch.reshape(10, 1, 1000)
    return _tc_pool(accP1, accP2, _row8(bp1), _row8(bp2),
                    Wp1pad, Wp2pad, bprj, batch3d)


# speedup vs baseline: 1.1508x; 1.1508x over previous
import functools

import jax
import jax.numpy as jnp
from jax import lax
from jax.experimental import pallas as pl
from jax.experimental.pallas import tpu as pltpu
from jax.experimental.pallas import tpu_sc as plsc

N = 10000
E = 160000
H = 256
G = 64

NSUB = 16
CHUNK = 64
NCHUNK = 160
SUPER = 40
EPS = CHUNK * NCHUNK
PADC = 144
ROWS_PER_SUB = 624
NEG = -1e30


def _mm_kernel(x_ref, w_ref, o_ref):
    o_ref[...] = jnp.dot(x_ref[...], w_ref[...],
                         preferred_element_type=jnp.float32)


def _mm(x, w, bm=2000):
    M, K = x.shape
    _, Nc = w.shape
    return pl.pallas_call(
        _mm_kernel,
        grid=(M // bm,),
        in_specs=[
            pl.BlockSpec((bm, K), lambda i: (i, 0)),
            pl.BlockSpec((K, Nc), lambda i: (0, 0)),
        ],
        out_specs=pl.BlockSpec((bm, Nc), lambda i: (i, 0)),
        out_shape=jax.ShapeDtypeStruct((M, Nc), jnp.float32),
    )(x, w)


def _pad16(bm, acc_sub, svec):
    it = lax.broadcasted_iota(jnp.int32, (bm, 16), 1)
    pad = jnp.where(it == 0, 1.0, 0.0) + jnp.where(it == 1, 1.0, 0.0) * svec
    return jnp.concatenate([acc_sub, pad], axis=1)


def _tcb_kernel(x_ref, w_ref, b_ref, lo_ref, hi_ref, sd_ref, *, bm):
    acc = jnp.dot(x_ref[...], w_ref[...],
                  preferred_element_type=jnp.float32) + b_ref[0:1, :]
    sv = acc[:, 256:257]
    lo_ref[...] = _pad16(bm, acc[:, 0:128], sv)
    hi_ref[...] = _pad16(bm, acc[:, 128:256], sv)
    sd_ref[...] = acc[:, 256:384]


def _tc_in(x, Wcomb, biasrow, bm=2000):
    M = x.shape[0]
    return pl.pallas_call(
        functools.partial(_tcb_kernel, bm=bm),
        grid=(M // bm,),
        in_specs=[
            pl.BlockSpec((bm, H), lambda i: (i, 0)),
            pl.BlockSpec((H, H + 128), lambda i: (0, 0)),
            pl.BlockSpec((8, H + 128), lambda i: (0, 0)),
        ],
        out_specs=[
            pl.BlockSpec((bm, PADC), lambda i: (i, 0)),
            pl.BlockSpec((bm, PADC), lambda i: (i, 0)),
            pl.BlockSpec((bm, 128), lambda i: (i, 0)),
        ],
        out_shape=[
            jax.ShapeDtypeStruct((M, PADC), jnp.float32),
            jax.ShapeDtypeStruct((M, PADC), jnp.float32),
            jax.ShapeDtypeStruct((M, 128), jnp.float32),
        ],
    )(x, Wcomb, biasrow)


def _normalize(a_ref):
    lo = a_ref[0, :, 0:128] / (a_ref[0, :, 128:129] + 1e-16)
    hi = a_ref[1, :, 0:128] / (a_ref[1, :, 128:129] + 1e-16)
    return jnp.concatenate([lo, hi], axis=1)


def _tcc_kernel(a_ref, bp_ref, w_ref, b_ref, lo_ref, hi_ref, sd_ref, *, bm):
    h = jax.nn.relu(_normalize(a_ref) + bp_ref[0:1, :])
    acc = jnp.dot(h, w_ref[...],
                  preferred_element_type=jnp.float32) + b_ref[0:1, :]
    sv = acc[:, 256:257]
    lo_ref[...] = _pad16(bm, acc[:, 0:128], sv)
    hi_ref[...] = _pad16(bm, acc[:, 128:256], sv)
    sd_ref[...] = acc[:, 256:384]


def _tcc2_kernel(a_ref, bp_ref, w_ref, lo1_ref, hi1_ref, sd1_ref,
                 lo2_ref, hi2_ref, sd2_ref, *, bm):
    h = jax.nn.relu(_normalize(a_ref) + bp_ref[0:1, :])
    acc = jnp.dot(h, w_ref[...], preferred_element_type=jnp.float32)
    sv1 = acc[:, 256:257]
    lo1_ref[...] = _pad16(bm, acc[:, 0:128], sv1)
    hi1_ref[...] = _pad16(bm, acc[:, 128:256], sv1)
    sd1_ref[...] = acc[:, 256:384]
    sv2 = acc[:, 640:641]
    lo2_ref[...] = _pad16(bm, acc[:, 384:512], sv2)
    hi2_ref[...] = _pad16(bm, acc[:, 512:640], sv2)
    sd2_ref[...] = acc[:, 640:768]


def _tc_mid2(acc, bprevrow, Wcomb2, bm=2000):
    outs = [
        jax.ShapeDtypeStruct((N, PADC), jnp.float32),
        jax.ShapeDtypeStruct((N, PADC), jnp.float32),
        jax.ShapeDtypeStruct((N, 128), jnp.float32),
    ] * 2
    return pl.pallas_call(
        functools.partial(_tcc2_kernel, bm=bm),
        grid=(N // bm,),
        in_specs=[
            pl.BlockSpec((2, bm, PADC), lambda i: (0, i, 0)),
            pl.BlockSpec((8, H), lambda i: (0, 0)),
            pl.BlockSpec((H, 768), lambda i: (0, 0)),
        ],
        out_specs=[pl.BlockSpec((bm, PADC), lambda i: (i, 0)),
                   pl.BlockSpec((bm, PADC), lambda i: (i, 0)),
                   pl.BlockSpec((bm, 128), lambda i: (i, 0))] * 2,
        out_shape=outs,
    )(acc, bprevrow, Wcomb2)


def _tc_mid(acc, bprevrow, Wcomb, biasrow, bm=2000):
    return pl.pallas_call(
        functools.partial(_tcc_kernel, bm=bm),
        grid=(N // bm,),
        in_specs=[
            pl.BlockSpec((2, bm, PADC), lambda i: (0, i, 0)),
            pl.BlockSpec((8, H), lambda i: (0, 0)),
            pl.BlockSpec((H, H + 128), lambda i: (0, 0)),
            pl.BlockSpec((8, H + 128), lambda i: (0, 0)),
        ],
        out_specs=[
            pl.BlockSpec((bm, PADC), lambda i: (i, 0)),
            pl.BlockSpec((bm, PADC), lambda i: (i, 0)),
            pl.BlockSpec((bm, 128), lambda i: (i, 0)),
        ],
        out_shape=[
            jax.ShapeDtypeStruct((N, PADC), jnp.float32),
            jax.ShapeDtypeStruct((N, PADC), jnp.float32),
            jax.ShapeDtypeStruct((N, 128), jnp.float32),
        ],
    )(acc, bprevrow, Wcomb, biasrow)


def _tcd_kernel(a1_ref, a2_ref, bp1_ref, bp2_ref, wp1_ref, wp2_ref,
                bprj_ref, batch_ref, o_ref, sums, *, bm, ng):
    i = pl.program_id(0)
    xg1 = _normalize(a1_ref) + bp1_ref[0:1, :]
    xg2 = _normalize(a2_ref) + bp2_ref[0:1, :]
    z1 = jnp.dot(xg1, wp1_ref[...],
                 preferred_element_type=jnp.float32)[:, 0:1] + bprj_ref[0:1, 0:1]
    z2 = jnp.dot(xg2, wp2_ref[...],
                 preferred_element_type=jnp.float32)[:, 0:1] + bprj_ref[0:1, 1:2]
    xw1 = jax.nn.sigmoid(z1) * xg1
    xw2 = jax.nn.sigmoid(z2) * xg2
    b = batch_ref[0, 0, :]
    oh = (b[:, None] == lax.broadcasted_iota(jnp.int32, (bm, G), 1)
          ).astype(jnp.float32)
    payload = jnp.concatenate(
        [xw1, xw2, jnp.ones((bm, 128), jnp.float32)], axis=1)
    contrib = lax.dot_general(oh, payload, (((0,), (0,)), ((), ())),
                              preferred_element_type=jnp.float32)

    @pl.when(i == 0)
    def _():
        sums[...] = jnp.zeros_like(sums)

    sums[...] += contrib

    @pl.when(i == ng - 1)
    def _():
        cnt = jnp.clip(sums[:, 512:513], 1.0)
        o_ref[...] = sums[:, 0:512] / cnt


def _tc_pool(acc1, acc2, bp1row, bp2row, Wp1pad, Wp2pad, bprjrow, batch3d,
             bm=1000):
    ng = N // bm
    return pl.pallas_call(
        functools.partial(_tcd_kernel, bm=bm, ng=ng),
        grid=(ng,),
        in_specs=[
            pl.BlockSpec((2, bm, PADC), lambda i: (0, i, 0)),
            pl.BlockSpec((2, bm, PADC), lambda i: (0, i, 0)),
            pl.BlockSpec((8, H), lambda i: (0, 0)),
            pl.BlockSpec((8, H), lambda i: (0, 0)),
            pl.BlockSpec((H, 128), lambda i: (0, 0)),
            pl.BlockSpec((H, 128), lambda i: (0, 0)),
            pl.BlockSpec((8, 128), lambda i: (0, 0)),
            pl.BlockSpec((1, 1, bm), lambda i: (i, 0, 0)),
        ],
        out_specs=pl.BlockSpec((G, 512), lambda i: (0, 0)),
        out_shape=jax.ShapeDtypeStruct((G, 512), jnp.float32),
        scratch_shapes=[pltpu.VMEM((G, 640), jnp.float32)],
    )(acc1, acc2, bp1row, bp2row, Wp1pad, Wp2pad, bprjrow, batch3d)


def _sc_body(lo, hi, d_hbm, src3d, dst3d, te3d, out,
             acc, dtab, srcc, dstc, tec, rows0, rows1, pb,
             g0, g1, s0, s1):
    c = lax.axis_index("c")
    s = lax.axis_index("s")

    def zrow(k, _):
        for j in range(PADC // 16):
            rows0[k, pl.ds(j * 16, 16)] = jnp.zeros((16,), jnp.float32)
        return 0
    lax.fori_loop(0, CHUNK, zrow, 0)
    base = s * ROWS_PER_SUB
    for k in range(9):
        pltpu.sync_copy(rows0, acc.at[pl.ds(base + k * CHUNK, CHUNK)])
    pltpu.sync_copy(rows0.at[pl.ds(0, 48)], acc.at[pl.ds(base + 576, 48)])

    @pl.when(s == NSUB - 1)
    def _():
        pltpu.sync_copy(rows0.at[pl.ds(0, 16)],
                        acc.at[pl.ds(NSUB * ROWS_PER_SUB, 16)])

    plsc.subcore_barrier()

    pltpu.sync_copy(d_hbm, dtab)

    def _gather(idx_ref, buf, sem):
        @pl.when(c == 0)
        def _():
            pltpu.async_copy(lo.at[idx_ref], buf, sem)

        @pl.when(c == 1)
        def _():
            pltpu.async_copy(hi.at[idx_ref], buf, sem)

    def _drain(buf, sem):
        pltpu.make_async_copy(lo.at[srcc.at[0]], buf, sem).wait()

    def _process(row, buf, gsem, ssem):
        _drain(buf, gsem)

        col129 = jnp.full((16,), 129, jnp.int32)
        for g in range(CHUNK // 16):
            sl = pl.ds(g * 16, 16)
            rowi = lax.iota(jnp.int32, 16) + g * 16
            sv = plsc.load_gather(buf, [rowi, col129])
            dv = plsc.load_gather(dtab, [dstc[row, sl]])
            alpha = sv + dv + tec[row, sl]
            alpha = jnp.where(alpha > 0, alpha, 0.2 * alpha)
            pb[sl] = jnp.exp(alpha)

        @plsc.parallel_loop(0, CHUNK, unroll=8)
        def _(e):
            pv = plsc.load_gather(pb, [jnp.full((16,), e, jnp.int32)])
            for j in range(PADC // 16):
                jsl = pl.ds(j * 16, 16)
                buf[e, jsl] = buf[e, jsl] * pv

        pltpu.async_copy(buf, acc.at[dstc.at[row]], ssem, add=True)

    QP = SUPER // 2

    def _drain_scatter(sem):
        pltpu.make_async_copy(rows0, acc.at[dstc.at[0]], sem).wait()

    def pair_body(p, _):
        q = p % QP

        @pl.when(q == 0)
        def _():
            @pl.when(p > 0)
            def _():
                _drain_scatter(s1)
            cb = (p // QP) * SUPER
            pltpu.sync_copy(src3d.at[s].at[pl.ds(cb, SUPER)], srcc)
            pltpu.sync_copy(dst3d.at[s].at[pl.ds(cb, SUPER)], dstc)
            pltpu.sync_copy(te3d.at[s].at[pl.ds(cb, SUPER)], tec)
            _gather(srcc.at[0], rows0, g0)

        @pl.when(q > 0)
        def _():
            _drain_scatter(s1)

        kk = 2 * q
        _gather(srcc.at[kk + 1], rows1, g1)
        _process(kk, rows0, g0, s0)
        _process(kk + 1, rows1, g1, s1)

        @pl.when(q < QP - 1)
        def _():
            _drain_scatter(s0)
            _gather(srcc.at[kk + 2], rows0, g0)

        @pl.when(q == QP - 1)
        def _():
            _drain_scatter(s0)
        return 0

    lax.fori_loop(0, NCHUNK // 2, pair_body, 0)
    _drain_scatter(s1)
    plsc.subcore_barrier()

    pltpu.sync_copy(acc.at[pl.ds(base, ROWS_PER_SUB)],
                    out.at[c].at[pl.ds(base, ROWS_PER_SUB)])

    @pl.when(s == NSUB - 1)
    def _():
        pltpu.sync_copy(acc.at[pl.ds(NSUB * ROWS_PER_SUB, 16)],
                        out.at[c].at[pl.ds(NSUB * ROWS_PER_SUB, 16)])


_sc_mesh = plsc.VectorSubcoreMesh(core_axis_name="c", subcore_axis_name="s")

_sc_gat = pl.kernel(
    _sc_body,
    out_type=jax.ShapeDtypeStruct((2, N, PADC), jnp.float32),
    mesh=_sc_mesh,
    compiler_params=pltpu.CompilerParams(use_tc_tiling_on_sc=False,
                                         needs_layout_passes=False),
    scratch_types=[
        pltpu.VMEM_SHARED((N, PADC), jnp.float32),
        pltpu.VMEM((N,), jnp.float32),
        pltpu.VMEM((SUPER, CHUNK), jnp.int32),
        pltpu.VMEM((SUPER, CHUNK), jnp.int32),
        pltpu.VMEM((SUPER, CHUNK), jnp.float32),
        pltpu.VMEM((CHUNK, PADC), jnp.float32),
        pltpu.VMEM((CHUNK, PADC), jnp.float32),
        pltpu.VMEM((CHUNK,), jnp.float32),
        pltpu.SemaphoreType.DMA,
        pltpu.SemaphoreType.DMA,
        pltpu.SemaphoreType.DMA,
        pltpu.SemaphoreType.DMA,
    ],
)


def _aug(W, a_s, a_d):
    Z = jnp.zeros((H, 128), jnp.float32)
    Z = Z.at[:, 0].set(W @ a_s)
    Z = Z.at[:, 1].set(W @ a_d)
    return jnp.concatenate([W, Z], axis=1)


def _row8(v):
    return jnp.broadcast_to(v[None, :], (8, v.shape[0]))


def _slab_f32(v):
    v2 = v.reshape(NSUB, N)
    v2 = jnp.concatenate(
        [v2, jnp.full((NSUB, EPS - N), NEG, jnp.float32)], axis=1)
    return v2.reshape(NSUB, NCHUNK, CHUNK)


def _slab_i32(v):
    v2 = v.reshape(NSUB, N)
    v2 = jnp.concatenate(
        [v2, jnp.zeros((NSUB, EPS - N), jnp.int32)], axis=1)
    return v2.reshape(NSUB, NCHUNK, CHUNK)


def kernel(x, edge_index, edge_attr, batch, W_np, b_np, W_ep, b_ep, W1, a_s1, a_d1, We1, a_e1, b1, W2, a_s2, a_d2, We2, a_e2, b2, Wp1, as_p1, ad_p1, bp1, Wproj1, bproj1, Wp2, as_p2, ad_p2, bp2, Wproj2, bproj2):
    src, dst = edge_index[0], edge_index[1]
    src2d = _slab_i32(src)
    dst2d = _slab_i32(dst)

    v1 = We1 @ a_e1
    v2 = We2 @ a_e2
    w1 = W_ep @ v1
    w2 = W_ep @ v2
    W8 = jnp.zeros((8, 16, 128), jnp.float32)
    for i in range(8):
        W8 = W8.at[i, :, 2 * i].set(w1).at[i, :, 2 * i + 1].set(w2)
    W8 = W8.reshape(128, 128)
    te_pack = _mm(edge_attr.reshape(E // 8, 128), W8)
    tt = te_pack[:, 0:16].reshape(E // 8, 8, 2)
    te1_2d = _slab_f32(tt[:, :, 0].reshape(E))
    te2_2d = _slab_f32(tt[:, :, 1].reshape(E))
    te0_2d = _slab_f32(jnp.zeros((E,), jnp.float32))

    W1aug = _aug(W1, a_s1, a_d1)
    Wcomb1 = W_np @ W1aug
    bias1 = (b_np @ W1aug).at[H + 1].add(b_ep @ v1)
    lo1, hi1, sd1 = _tc_in(x, Wcomb1, _row8(bias1))
    acc1 = _sc_gat(lo1, hi1, sd1[:, 1], src2d, dst2d, te1_2d)

    bias2 = jnp.zeros((H + 128,), jnp.float32).at[H + 1].add(b_ep @ v2)
    lo2, hi2, sd2 = _tc_mid(acc1, _row8(b1), _aug(W2, a_s2, a_d2),
                            _row8(bias2))
    acc2 = _sc_gat(lo2, hi2, sd2[:, 1], src2d, dst2d, te2_2d)

    Wcomb2 = jnp.concatenate(
        [_aug(Wp1, as_p1, ad_p1), _aug(Wp2, as_p2, ad_p2)], axis=1)
    loP1, hiP1, sdP1, loP2, hiP2, sdP2 = _tc_mid2(acc2, _row8(b2), Wcomb2)
    accP1 = _sc_gat(loP1, hiP1, sdP1[:, 1], src2d, dst2d, te0_2d)
    accP2 = _sc_gat(loP2, hiP2, sdP2[:, 1], src2d, dst2d, te0_2d)

    Wp1pad = jnp.zeros((H, 128), jnp.float32).at[:, 0].set(Wproj1[:, 0])
    Wp2pad = jnp.zeros((H, 128), jnp.float32).at[:, 0].set(Wproj2[:, 0])
    bprj = jnp.zeros((8, 128), jnp.float32)
    bprj = bprj.at[:, 0].set(bproj1[0]).at[:, 1].set(bproj2[0])
    batch3d = batch.reshape(10, 1, 1000)
    return _tc_pool(accP1, accP2, _row8(bp1), _row8(bp2),
                    Wp1pad, Wp2pad, bprj, batch3d)
